# Initial kernel scaffold; baseline (speedup 1.0000x reference)
import jax
import jax.numpy as jnp
from jax.experimental import pallas as pl


def kernel(x, edge_index, edge_attr, W1, b1, g1, be1, W2, b2, g2, be2):
    raise NotImplementedError("write your pallas kernel here")


# speedup vs baseline: 89.3438x; 89.3438x over previous
import dataclasses
import functools

import jax
import jax.numpy as jnp
from jax import lax
from jax.experimental import pallas as pl
from jax.experimental.pallas import tpu as pltpu
from jax.experimental.pallas import tpu_sc as plsc

_N = 10000
_E = 320000
_NC = 2
_NS = 16
_NW = _NC * _NS
_L = 16
_NPAD = 10240
_NPT = _NPAD // _NS
_RPW = 80
_EPAD = _NW * _RPW * 128
_EROWS = _EPAD // 128

_mesh = plsc.VectorSubcoreMesh(
    core_axis_name="c", subcore_axis_name="s",
    num_cores=_NC, num_subcores=_NS)

_sc_params = pltpu.CompilerParams()
if "needs_layout_passes" in pltpu.CompilerParams.__dataclass_fields__:
    _sc_params = dataclasses.replace(_sc_params, needs_layout_passes=False)


def _zero_fill(z_v):
    @pl.loop(0, _NPT, step=_L)
    def _zero(i):
        z_v[pl.ds(i, _L)] = jnp.zeros((_L,), jnp.float32)


@functools.partial(
    pl.kernel,
    out_type=jax.ShapeDtypeStruct((_NC, _NPAD), jnp.float32),
    mesh=_mesh,
    compiler_params=_sc_params,
    scratch_types=[
        pltpu.VMEM((_RPW, 128), jnp.int32),
        pltpu.VMEM((_RPW, 128), jnp.float32),
        pltpu.VMEM((_NPT,), jnp.float32),
        pltpu.VMEM_SHARED((_NPAD,), jnp.float32),
    ],
)
def _sc_degree(dst_hbm, w_hbm, out_hbm, dst_v, w_v, z_v, deg_s):
    c = lax.axis_index("c")
    s = lax.axis_index("s")
    wid = c * _NS + s

    _zero_fill(z_v)
    pltpu.sync_copy(z_v, deg_s.at[pl.ds(s * _NPT, _NPT)])
    pltpu.sync_copy(dst_hbm.at[pl.ds(wid * _RPW, _RPW)], dst_v)
    pltpu.sync_copy(w_hbm.at[pl.ds(wid * _RPW, _RPW)], w_v)
    plsc.subcore_barrier()

    @pl.loop(0, _RPW)
    def _scat(r):
        pltpu.sync_copy(w_v.at[r], deg_s.at[dst_v.at[r]], add=True)

    plsc.subcore_barrier()

    @pl.when(s == 0)
    def _out():
        pltpu.sync_copy(deg_s, out_hbm.at[c])


@functools.partial(
    pl.kernel,
    out_type=[jax.ShapeDtypeStruct((_NC, 2, _NPAD), jnp.float32),
              jax.ShapeDtypeStruct((_EROWS, 128), jnp.float32)],
    mesh=_mesh,
    compiler_params=_sc_params,
    scratch_types=[
        pltpu.VMEM((_NPAD,), jnp.float32),
        pltpu.VMEM((_NPAD,), jnp.float32),
        pltpu.VMEM((_NPAD,), jnp.float32),
        pltpu.VMEM((_RPW, 128), jnp.int32),
        pltpu.VMEM((_RPW, 128), jnp.int32),
        pltpu.VMEM((_RPW, 128), jnp.float32),
        pltpu.VMEM((_RPW, 128), jnp.float32),
        pltpu.VMEM((_RPW, 128), jnp.float32),
        pltpu.VMEM((_RPW, 128), jnp.float32),
        pltpu.VMEM((_NPT,), jnp.float32),
        pltpu.VMEM_SHARED((_NPAD,), jnp.float32),
        pltpu.VMEM_SHARED((_NPAD,), jnp.float32),
    ],
)
def _sc_layer1(src_hbm, dst_hbm, w_hbm, dis_hbm, xwt_hbm, h1p_hbm, norm_hbm,
               dis_v, xw0_v, xw1_v, src_v, dst_v, w_v, norm_v, val0_v, val1_v,
               z_v, acc0_s, acc1_s):
    c = lax.axis_index("c")
    s = lax.axis_index("s")
    wid = c * _NS + s
    rowbase = wid * _RPW

    _zero_fill(z_v)
    pltpu.sync_copy(z_v, acc0_s.at[pl.ds(s * _NPT, _NPT)])
    pltpu.sync_copy(z_v, acc1_s.at[pl.ds(s * _NPT, _NPT)])
    pltpu.sync_copy(dis_hbm, dis_v)
    pltpu.sync_copy(xwt_hbm.at[0], xw0_v)
    pltpu.sync_copy(xwt_hbm.at[1], xw1_v)
    pltpu.sync_copy(src_hbm.at[pl.ds(rowbase, _RPW)], src_v)
    pltpu.sync_copy(dst_hbm.at[pl.ds(rowbase, _RPW)], dst_v)
    pltpu.sync_copy(w_hbm.at[pl.ds(rowbase, _RPW)], w_v)
    plsc.subcore_barrier()

    @pl.loop(0, _RPW)
    def _compute(r):
        for j in range(128 // _L):
            sl = pl.ds(j * _L, _L)
            si = src_v[r, sl]
            di = dst_v[r, sl]
            wv = w_v[r, sl]
            nrm = (plsc.load_gather(dis_v, [si]) * wv
                   * plsc.load_gather(dis_v, [di]))
            norm_v[r, sl] = nrm
            val0_v[r, sl] = plsc.load_gather(xw0_v, [si]) * nrm
            val1_v[r, sl] = plsc.load_gather(xw1_v, [si]) * nrm

    pltpu.sync_copy(norm_v, norm_hbm.at[pl.ds(rowbase, _RPW)])

    @pl.loop(0, _RPW)
    def _scatter(r):
        pltpu.sync_copy(val0_v.at[r], acc0_s.at[dst_v.at[r]], add=True)
        pltpu.sync_copy(val1_v.at[r], acc1_s.at[dst_v.at[r]], add=True)

    plsc.subcore_barrier()

    @pl.when(s == 0)
    def _out():
        pltpu.sync_copy(acc0_s, h1p_hbm.at[c, 0])
        pltpu.sync_copy(acc1_s, h1p_hbm.at[c, 1])


@functools.partial(
    pl.kernel,
    out_type=jax.ShapeDtypeStruct((_NC, _NPAD), jnp.float32),
    mesh=_mesh,
    compiler_params=_sc_params,
    scratch_types=[
        pltpu.VMEM((_NPAD,), jnp.float32),
        pltpu.VMEM((_RPW, 128), jnp.int32),
        pltpu.VMEM((_RPW, 128), jnp.int32),
        pltpu.VMEM((_RPW, 128), jnp.float32),
        pltpu.VMEM((_RPW, 128), jnp.float32),
        pltpu.VMEM((_NPT,), jnp.float32),
        pltpu.VMEM_SHARED((_NPAD,), jnp.float32),
    ],
)
def _sc_layer2(src_hbm, dst_hbm, norm_hbm, hw2_hbm, out_hbm,
               hw2_v, src_v, dst_v, norm_v, val_v, z_v, acc_s):
    c = lax.axis_index("c")
    s = lax.axis_index("s")
    wid = c * _NS + s
    rowbase = wid * _RPW

    _zero_fill(z_v)
    pltpu.sync_copy(z_v, acc_s.at[pl.ds(s * _NPT, _NPT)])
    pltpu.sync_copy(hw2_hbm, hw2_v)
    pltpu.sync_copy(src_hbm.at[pl.ds(rowbase, _RPW)], src_v)
    pltpu.sync_copy(dst_hbm.at[pl.ds(rowbase, _RPW)], dst_v)
    pltpu.sync_copy(norm_hbm.at[pl.ds(rowbase, _RPW)], norm_v)
    plsc.subcore_barrier()

    @pl.loop(0, _RPW)
    def _compute(r):
        for j in range(128 // _L):
            sl = pl.ds(j * _L, _L)
            si = src_v[r, sl]
            val_v[r, sl] = plsc.load_gather(hw2_v, [si]) * norm_v[r, sl]

    @pl.loop(0, _RPW)
    def _scatter(r):
        pltpu.sync_copy(val_v.at[r], acc_s.at[dst_v.at[r]], add=True)

    plsc.subcore_barrier()

    @pl.when(s == 0)
    def _out():
        pltpu.sync_copy(acc_s, out_hbm.at[c])


def _tc_xw_body(x_ref, w1_ref, xwt_ref):
    xwt_ref[...] = lax.dot_general(
        w1_ref[...], x_ref[...],
        dimension_numbers=(((1,), (1,)), ((), ())),
        preferred_element_type=jnp.float32)


_tc_xw = pl.pallas_call(
    _tc_xw_body,
    out_shape=jax.ShapeDtypeStruct((2, _NPAD), jnp.float32),
)


def _tc_dis_body(degp_ref, dis_ref):
    deg = degp_ref[0:1, :] + degp_ref[1:2, :] + 1.0
    dis_ref[...] = lax.rsqrt(deg)


_tc_dis = pl.pallas_call(
    _tc_dis_body,
    out_shape=jax.ShapeDtypeStruct((1, _NPAD), jnp.float32),
)


def _leaky(h):
    return jnp.where(h >= 0, h, 0.1 * h)


def _tc_mid_body(h1p_ref, xwt_ref, dis_ref, p_ref, hw2_ref):
    dis2 = dis_ref[...] * dis_ref[...]
    col = lax.broadcasted_iota(jnp.int32, (1, _NPAD), 1)
    msk = col < _N
    inv_n = 1.0 / _N
    hn = []
    for ch in range(2):
        h = (h1p_ref[ch:ch + 1, :] + h1p_ref[2 + ch:3 + ch, :]
             + xwt_ref[ch:ch + 1, :] * dis2 + p_ref[ch])
        h = _leaky(h)
        mean = jnp.sum(jnp.where(msk, h, 0.0)) * inv_n
        var = jnp.sum(jnp.where(msk, (h - mean) * (h - mean), 0.0)) * inv_n
        hn.append((h - mean) * lax.rsqrt(var + 1e-5) * p_ref[2 + ch]
                  + p_ref[4 + ch])
    hw2 = hn[0] * p_ref[6] + hn[1] * p_ref[7]
    hw2_ref[...] = jnp.where(msk, hw2, 0.0)


_tc_mid = pl.pallas_call(
    _tc_mid_body,
    in_specs=[
        pl.BlockSpec(memory_space=pltpu.VMEM),
        pl.BlockSpec(memory_space=pltpu.VMEM),
        pl.BlockSpec(memory_space=pltpu.VMEM),
        pl.BlockSpec(memory_space=pltpu.SMEM),
    ],
    out_shape=jax.ShapeDtypeStruct((1, _NPAD), jnp.float32),
)


def _tc_fin_body(h2p_ref, hw2_ref, dis_ref, p_ref, out_ref):
    dis2 = dis_ref[...] * dis_ref[...]
    col = lax.broadcasted_iota(jnp.int32, (1, _NPAD), 1)
    msk = col < _N
    inv_n = 1.0 / _N
    h = (h2p_ref[0:1, :] + h2p_ref[1:2, :] + hw2_ref[...] * dis2 + p_ref[0])
    h = _leaky(h)
    mean = jnp.sum(jnp.where(msk, h, 0.0)) * inv_n
    var = jnp.sum(jnp.where(msk, (h - mean) * (h - mean), 0.0)) * inv_n
    out_ref[...] = (h - mean) * lax.rsqrt(var + 1e-5) * p_ref[1] + p_ref[2]


_tc_fin = pl.pallas_call(
    _tc_fin_body,
    in_specs=[
        pl.BlockSpec(memory_space=pltpu.VMEM),
        pl.BlockSpec(memory_space=pltpu.VMEM),
        pl.BlockSpec(memory_space=pltpu.VMEM),
        pl.BlockSpec(memory_space=pltpu.SMEM),
    ],
    out_shape=jax.ShapeDtypeStruct((1, _NPAD), jnp.float32),
)


def kernel(x, edge_index, edge_attr, W1, b1, g1, be1, W2, b2, g2, be2):
    src = edge_index[0]
    dst = edge_index[1]
    pad = _EPAD - _E
    pad_dst = (_N + jnp.arange(pad, dtype=jnp.int32) % (_NPAD - _N))
    pad_src = jnp.arange(pad, dtype=jnp.int32) % _N
    srcp = jnp.concatenate([src, pad_src]).reshape(_EROWS, 128)
    dstp = jnp.concatenate([dst, pad_dst]).reshape(_EROWS, 128)
    wp = jnp.pad(edge_attr, (0, pad)).reshape(_EROWS, 128)
    xpad = jnp.pad(x, ((0, _NPAD - _N), (0, 0)))

    degp = _sc_degree(dstp, wp)
    xwt = _tc_xw(xpad, W1)
    dis = _tc_dis(degp)
    h1p, normp = _sc_layer1(srcp, dstp, wp, dis.reshape(_NPAD), xwt)
    p1 = jnp.concatenate([b1, g1, be1, W2[0]])
    hw2 = _tc_mid(h1p.reshape(4, _NPAD), xwt, dis, p1)
    h2p = _sc_layer2(srcp, dstp, normp, hw2.reshape(_NPAD))
    p2 = jnp.concatenate([b2, g2, be2])
    out = _tc_fin(h2p, hw2, dis, p2)
    return out[0, :_N].reshape(_N, 1)
